# Initial kernel scaffold; baseline (speedup 1.0000x reference)
import jax
import jax.numpy as jnp
from jax.experimental import pallas as pl


def kernel(node_feat, node_config_feat, node_opcode, edge_index, batch, op_emb, shape_emb, lin_W, lin_b, l0_Wp, l0_bp, l0_Wl, l0_bl, l0_Wr, l1_Wp, l1_bp, l1_Wl, l1_bl, l1_Wr, l2_Wp, l2_bp, l2_Wl, l2_bl, l2_Wr, post_W, post_b):
    raise NotImplementedError("write your pallas kernel here")


# speedup vs baseline: 2.8111x; 2.8111x over previous
import functools

import jax
import jax.numpy as jnp
from jax import lax
from jax.experimental import pallas as pl
from jax.experimental.pallas import tpu as pltpu
from jax.experimental.pallas import tpu_sc as plsc

N = 50000
E = 800000
H = 64
NG = 16
N_OPS = 120

NSUB = 16
CH = 128
NCHUNK = E // CH
ACC_ROWS = 16 * 1664
ZSTRIPE = ACC_ROWS // NSUB
HALF0 = 25088
HALF1 = N - HALF0
TRASH = ACC_ROWS
DEGW = 16

@functools.lru_cache(maxsize=None)
def _sc_mesh():
    return plsc.VectorSubcoreMesh(core_axis_name="c", subcore_axis_name="s",
                                  num_cores=2, num_subcores=NSUB)


def _localize_dst(dst_v, base, half):
    for j in range(CH // 16):
        d = dst_v[pl.ds(j * 16, 16)]
        loc = d - base
        ok = (loc >= 0) & (loc < half)
        dst_v[pl.ds(j * 16, 16)] = jnp.where(ok, loc, TRASH)


def _writeback(acc, out_hbm, c, s):
    @pl.when(c == 0)
    def _():
        pltpu.sync_copy(acc.at[pl.ds(s * 1568, 1568)],
                        out_hbm.at[pl.ds(s * 1568, 1568)])

    @pl.when((c == 1) & (s < 15))
    def _():
        pltpu.sync_copy(acc.at[pl.ds(s * 1560, 1560)],
                        out_hbm.at[pl.ds(HALF0 + s * 1560, 1560)])

    @pl.when((c == 1) & (s == 15))
    def _():
        pltpu.sync_copy(acc.at[pl.ds(15 * 1560, 1512)],
                        out_hbm.at[pl.ds(HALF0 + 15 * 1560, 1512)])


def _sc_aggregate(y, src, dst):
    @functools.partial(
        pl.kernel, mesh=_sc_mesh(),
        out_type=jax.ShapeDtypeStruct((N, H), jnp.float32),
        compiler_params=pltpu.CompilerParams(use_tc_tiling_on_sc=False),
        scratch_types=[
            pltpu.VMEM((CH,), jnp.int32),
            pltpu.VMEM((CH,), jnp.int32),
            pltpu.VMEM((CH, H), jnp.float32),
            pltpu.VMEM_SHARED((TRASH + 8, H), jnp.float32),
            pltpu.SemaphoreType.DMA,
        ])
    def agg(y_hbm, src_hbm, dst_hbm, out_hbm, src_v, dst_v, rows_v, acc, sem):
        c = lax.axis_index("c")
        s = lax.axis_index("s")
        base = c * HALF0
        half = jnp.where(c == 0, HALF0, HALF1)
        zero16 = jnp.zeros((16,), jnp.float32)

        @pl.loop(0, CH)
        def _(r):
            for j in range(H // 16):
                rows_v[r, pl.ds(j * 16, 16)] = zero16

        @pl.loop(0, ZSTRIPE // CH)
        def _(k):
            pltpu.sync_copy(rows_v, acc.at[pl.ds(s * ZSTRIPE + k * CH, CH)])

        @pl.when(s == 0)
        def _():
            pltpu.sync_copy(rows_v.at[pl.ds(0, 8)], acc.at[pl.ds(ACC_ROWS, 8)])

        plsc.subcore_barrier()

        nch = 390 + jnp.where(s < 10, 1, 0)

        @pl.loop(0, nch)
        def _(i):
            off = (s + i * NSUB) * CH
            pltpu.sync_copy(src_hbm.at[pl.ds(off, CH)], src_v)
            pltpu.sync_copy(dst_hbm.at[pl.ds(off, CH)], dst_v)
            _localize_dst(dst_v, base, half)
            pltpu.async_copy(y_hbm.at[src_v], rows_v, sem).wait()
            pltpu.sync_copy(rows_v, acc.at[dst_v], add=True)

        plsc.subcore_barrier()
        _writeback(acc, out_hbm, c, s)

    return agg(y, src, dst)


def _sc_degree(dst, ones_arr):
    @functools.partial(
        pl.kernel, mesh=_sc_mesh(),
        out_type=jax.ShapeDtypeStruct((N, DEGW), jnp.float32),
        compiler_params=pltpu.CompilerParams(use_tc_tiling_on_sc=False),
        scratch_types=[
            pltpu.VMEM((CH,), jnp.int32),
            pltpu.VMEM((CH, DEGW), jnp.float32),
            pltpu.VMEM((CH, DEGW), jnp.float32),
            pltpu.VMEM_SHARED((TRASH + 8, DEGW), jnp.float32),
        ])
    def deg(dst_hbm, ones_hbm, out_hbm, dst_v, ones_v, zeros_v, acc):
        c = lax.axis_index("c")
        s = lax.axis_index("s")
        base = c * HALF0
        half = jnp.where(c == 0, HALF0, HALF1)
        zero16 = jnp.zeros((16,), jnp.float32)

        pltpu.sync_copy(ones_hbm, ones_v)

        @pl.loop(0, CH)
        def _(r):
            zeros_v[r, pl.ds(0, 16)] = zero16

        @pl.loop(0, ZSTRIPE // CH)
        def _(k):
            pltpu.sync_copy(zeros_v, acc.at[pl.ds(s * ZSTRIPE + k * CH, CH)])

        @pl.when(s == 0)
        def _():
            pltpu.sync_copy(zeros_v.at[pl.ds(0, 8)], acc.at[pl.ds(ACC_ROWS, 8)])

        plsc.subcore_barrier()

        nch = 390 + jnp.where(s < 10, 1, 0)

        @pl.loop(0, nch)
        def _(i):
            off = (s + i * NSUB) * CH
            pltpu.sync_copy(dst_hbm.at[pl.ds(off, CH)], dst_v)
            _localize_dst(dst_v, base, half)
            pltpu.sync_copy(ones_v, acc.at[dst_v], add=True)

        plsc.subcore_barrier()
        _writeback(acc, out_hbm, c, s)

    return deg(dst, ones_arr)


R = 400
NBLK = N // R

_F32 = jnp.float32


def _dot(a, b):
    return jnp.dot(a, b, preferred_element_type=_F32)


def _full(shape):
    return pl.BlockSpec(shape, lambda i: (0,) * len(shape))


def _rows(width):
    return pl.BlockSpec((R, width), lambda i: (i, 0))


def _tc_prologue(feat128, cfg, aux, W_feat, W_cfg, shape_emb, W_set,
                 op_emb, W_op, bias):
    def body(f_ref, c_ref, a_ref, wf_ref, wc_ref, se_ref, ws_ref, oe_ref,
             wo_ref, b_ref, o_ref):
        x = _dot(f_ref[...], wf_ref[...])
        x += _dot(c_ref[...], wc_ref[...])
        Se = _dot(se_ref[...], ws_ref[...])
        Oe = _dot(oe_ref[...], wo_ref[...])
        aux_v = a_ref[...]
        sidx = aux_v[:, 0:1].astype(jnp.int32)
        oidx = aux_v[:, 1:2].astype(jnp.int32)
        oh_s = (lax.broadcasted_iota(jnp.int32, (R, 8), 1) == sidx).astype(_F32)
        x += _dot(oh_s, Se)
        oh_o = (lax.broadcasted_iota(jnp.int32, (R, N_OPS), 1) == oidx).astype(_F32)
        x += _dot(oh_o, Oe)
        o_ref[...] = jnp.maximum(x + b_ref[...], 0.0)

    return pl.pallas_call(
        body,
        grid=(NBLK,),
        in_specs=[_rows(128), _rows(18), _rows(8), _full((128, 128)),
                  _full((18, 128)), _full((8, 4)), _full((4, 128)),
                  _full((N_OPS, 32)), _full((32, 128)), _full((1, 128))],
        out_specs=_rows(128),
        out_shape=jax.ShapeDtypeStruct((N, 128), _F32),
    )(feat128, cfg, aux, W_feat, W_cfg, shape_emb, W_set, op_emb, W_op, bias)


def _tc_dense1(x, Wp, bp, Wl, ind):
    def body(x_ref, wp_ref, bp_ref, wl_ref, o_ref):
        xp = jnp.maximum(_dot(x_ref[...], wp_ref[...]) + bp_ref[...], 0.0)
        o_ref[...] = _dot(xp, wl_ref[...])

    return pl.pallas_call(
        body,
        grid=(NBLK,),
        in_specs=[_rows(ind), _full((ind, ind)), _full((1, ind)),
                  _full((ind, H))],
        out_specs=_rows(H),
        out_shape=jax.ShapeDtypeStruct((N, H), _F32),
    )(x, Wp, bp, Wl)


def _tc_dense2(s, degs, x, Wr, bl, ind):
    def body(s_ref, d_ref, x_ref, wr_ref, bl_ref, o_ref):
        cnt = jnp.maximum(d_ref[...][:, 0:1], 1.0)
        o = s_ref[...] / cnt + bl_ref[...] + _dot(x_ref[...], wr_ref[...])
        nrm = jnp.sqrt(jnp.sum(o * o, axis=1, keepdims=True))
        o_ref[...] = o / jnp.maximum(nrm, 1e-12)

    return pl.pallas_call(
        body,
        grid=(NBLK,),
        in_specs=[_rows(H), _rows(DEGW), _rows(ind), _full((ind, H)),
                  _full((1, H))],
        out_specs=_rows(H),
        out_shape=jax.ShapeDtypeStruct((N, H), _F32),
    )(s, degs, x, Wr, bl)


def _tc_pool(x, aux, pp):
    NEG = -3.4e38

    def body(x_ref, a_ref, pp_ref, o_ref, mx_ref, sm_ref, ct_ref):
        i = pl.program_id(0)

        @pl.when(i == 0)
        def _():
            mx_ref[...] = jnp.full((NG, H), NEG, _F32)
            sm_ref[...] = jnp.zeros((NG, H), _F32)
            ct_ref[...] = jnp.zeros((NG, 128), _F32)

        xv = x_ref[...]
        bidx = a_ref[...][:, 2:3].astype(jnp.int32)
        oh = (lax.broadcasted_iota(jnp.int32, (R, NG), 1) == bidx).astype(_F32)
        sm_ref[...] += lax.dot_general(oh, xv, (((0,), (0,)), ((), ())),
                                       preferred_element_type=_F32)
        ct_ref[...] += lax.dot_general(oh, jnp.ones((R, 128), _F32),
                                       (((0,), (0,)), ((), ())),
                                       preferred_element_type=_F32)
        for g in range(NG):
            m = bidx == g
            xm = jnp.where(m, xv, NEG)
            gm = jnp.max(xm, axis=0, keepdims=True)
            mx_ref[g:g + 1, :] = jnp.maximum(mx_ref[g:g + 1, :], gm)

        @pl.when(i == NBLK - 1)
        def _():
            cnt = jnp.maximum(ct_ref[...][:, 0:1], 1.0)
            xg = mx_ref[...] + sm_ref[...] / cnt
            nrm = jnp.sqrt(jnp.sum(xg * xg, axis=1, keepdims=True))
            xg = xg / nrm
            wt = pp_ref[...][0:1, 0:H]
            pb = pp_ref[...][1:2, 0:1]
            res = jnp.sum(xg * wt, axis=1, keepdims=True) + pb
            o_ref[...] = jnp.broadcast_to(res, (NG, 128))

    return pl.pallas_call(
        body,
        grid=(NBLK,),
        in_specs=[_rows(H), _rows(8), _full((8, 128))],
        out_specs=pl.BlockSpec((NG, 128), lambda i: (0, 0)),
        out_shape=jax.ShapeDtypeStruct((NG, 128), _F32),
        scratch_shapes=[pltpu.VMEM((NG, H), _F32), pltpu.VMEM((NG, H), _F32),
                        pltpu.VMEM((NG, 128), _F32)],
    )(x, aux, pp)


def kernel(node_feat, node_config_feat, node_opcode, edge_index, batch,
           op_emb, shape_emb, lin_W, lin_b,
           l0_Wp, l0_bp, l0_Wl, l0_bl, l0_Wr,
           l1_Wp, l1_bp, l1_Wl, l1_bl, l1_Wr,
           l2_Wp, l2_bp, l2_Wl, l2_bl, l2_Wr,
           post_W, post_b):
    src = edge_index[0]
    dst = edge_index[1]
    feat128 = node_feat[:, :128]
    aux = jnp.concatenate([
        node_feat[:, 128:129],
        node_opcode.astype(_F32)[:, None],
        batch.astype(_F32)[:, None],
        jnp.zeros((N, 5), _F32),
    ], axis=1)
    W_feat = lin_W[0:128]
    W_set = lin_W[128:132]
    W_op = lin_W[132:164]
    W_cfg = lin_W[164:182]
    bias = lin_b[None, :]
    pp = jnp.zeros((8, 128), _F32)
    pp = pp.at[0, 0:H].set(post_W[:, 0])
    pp = pp.at[1, 0].set(post_b[0])

    ones_arr = jnp.zeros((CH, DEGW), _F32).at[:, 0].set(1.0)
    degs = _sc_degree(dst, ones_arr)
    x = _tc_prologue(feat128, node_config_feat, aux, W_feat, W_cfg,
                     shape_emb, W_set, op_emb, W_op, bias)

    for Wp, bp, Wl, bl, Wr, ind in (
            (l0_Wp, l0_bp, l0_Wl, l0_bl, l0_Wr, 128),
            (l1_Wp, l1_bp, l1_Wl, l1_bl, l1_Wr, 64),
            (l2_Wp, l2_bp, l2_Wl, l2_bl, l2_Wr, 64)):
        y = _tc_dense1(x, Wp, bp[None, :], Wl, ind)
        s = _sc_aggregate(y, src, dst)
        x = _tc_dense2(s, degs, x, Wr, bl[None, :], ind)

    pooled = _tc_pool(x, aux, pp)
    return pooled[:, :1]


# speedup vs baseline: 3.6971x; 1.3152x over previous
import functools

import jax
import jax.numpy as jnp
from jax import lax
from jax.experimental import pallas as pl
from jax.experimental.pallas import tpu as pltpu
from jax.experimental.pallas import tpu_sc as plsc

N = 50000
E = 800000
H = 64
NG = 16
N_OPS = 120

NSUB = 16
CH = 128
EPC = 128
SUB = EPC // CH
NCHB = 391
EP = NSUB * NCHB * EPC
EROWS = (EP + EPC) // CH
ACC_ROWS = 16 * 1600
ZSTRIPE = ACC_ROWS // NSUB
HALF0 = 25088
HALF1 = N - HALF0
TRASH = ACC_ROWS
DEGW = 16

@functools.lru_cache(maxsize=None)
def _sc_mesh():
    return plsc.VectorSubcoreMesh(core_axis_name="c", subcore_axis_name="s",
                                  num_cores=2, num_subcores=NSUB)


def _localize_dst(dst_v, base, half):
    for k in range(SUB):
        for j in range(CH // 16):
            d = dst_v[k, pl.ds(j * 16, 16)]
            loc = d - base
            ok = (loc >= 0) & (loc < half)
            dst_v[k, pl.ds(j * 16, 16)] = jnp.where(ok, loc, TRASH)


def _writeback(acc, out_hbm, c, s):
    @pl.when(c == 0)
    def _():
        pltpu.sync_copy(acc.at[pl.ds(s * 1568, 1568)],
                        out_hbm.at[pl.ds(s * 1568, 1568)])

    @pl.when((c == 1) & (s < 15))
    def _():
        pltpu.sync_copy(acc.at[pl.ds(s * 1560, 1560)],
                        out_hbm.at[pl.ds(HALF0 + s * 1560, 1560)])

    @pl.when((c == 1) & (s == 15))
    def _():
        pltpu.sync_copy(acc.at[pl.ds(15 * 1560, 1512)],
                        out_hbm.at[pl.ds(HALF0 + 15 * 1560, 1512)])


def _sc_aggregate(y, src2d, dst2d):
    @functools.partial(
        pl.kernel, mesh=_sc_mesh(),
        out_type=jax.ShapeDtypeStruct((N, H), jnp.float32),
        compiler_params=pltpu.CompilerParams(use_tc_tiling_on_sc=False),
        scratch_types=[
            pltpu.VMEM((SUB, CH), jnp.int32),
            pltpu.VMEM((SUB, CH), jnp.int32),
            pltpu.VMEM((SUB, CH), jnp.int32),
            pltpu.VMEM((SUB, CH), jnp.int32),
            pltpu.VMEM((SUB, CH), jnp.int32),
            pltpu.VMEM((SUB, CH), jnp.int32),
            pltpu.VMEM((EPC, H), jnp.float32),
            pltpu.VMEM((EPC, H), jnp.float32),
            pltpu.VMEM((EPC, H), jnp.float32),
            pltpu.VMEM_SHARED((TRASH + 8, H), jnp.float32),
            pltpu.SemaphoreType.DMA,
            pltpu.SemaphoreType.DMA,
            pltpu.SemaphoreType.DMA,
            pltpu.SemaphoreType.DMA,
            pltpu.SemaphoreType.DMA,
            pltpu.SemaphoreType.DMA,
        ])
    def agg(y_hbm, src_hbm, dst_hbm, out_hbm,
            s0, s1, s2, d0, d1, d2, r0, r1, r2, acc,
            si0, si1, si2, sg0, sg1, sg2):
        c = lax.axis_index("c")
        s = lax.axis_index("s")
        base = c * HALF0
        half = jnp.where(c == 0, HALF0, HALF1)
        srcb, dstb, rowb = (s0, s1, s2), (d0, d1, d2), (r0, r1, r2)
        semi, semg = (si0, si1, si2), (sg0, sg1, sg2)
        zero16 = jnp.zeros((16,), jnp.float32)

        @pl.loop(0, EPC)
        def _(r):
            for j in range(H // 16):
                r0[r, pl.ds(j * 16, 16)] = zero16

        for kk in range(ZSTRIPE // EPC):
            pltpu.sync_copy(r0, acc.at[pl.ds(s * ZSTRIPE + kk * EPC, EPC)])
        _zrem = ZSTRIPE - (ZSTRIPE // EPC) * EPC
        if _zrem:
            pltpu.sync_copy(r0.at[pl.ds(0, _zrem)],
                            acc.at[pl.ds(s * ZSTRIPE + ZSTRIPE - _zrem, _zrem)])

        @pl.when(s == 0)
        def _():
            pltpu.sync_copy(r0.at[pl.ds(0, 8)], acc.at[pl.ds(ACC_ROWS, 8)])

        plsc.subcore_barrier()

        def row0(cc):
            return (s * NCHB + cc) * SUB

        def issue_idx(cc, b):
            pltpu.async_copy(src_hbm.at[pl.ds(row0(cc), SUB)], srcb[b], semi[b])
            pltpu.async_copy(dst_hbm.at[pl.ds(row0(cc), SUB)], dstb[b], semi[b])

        def wait_idx(cc, b):
            pltpu.make_async_copy(src_hbm.at[pl.ds(row0(cc), SUB)], srcb[b],
                                  semi[b]).wait()
            pltpu.make_async_copy(dst_hbm.at[pl.ds(row0(cc), SUB)], dstb[b],
                                  semi[b]).wait()

        def issue_gathers(b):
            for k in range(SUB):
                pltpu.async_copy(y_hbm.at[srcb[b].at[k]],
                                 rowb[b].at[pl.ds(k * CH, CH)], semg[b])

        def wait_gathers(b):
            for k in range(SUB):
                pltpu.make_async_copy(y_hbm.at[srcb[b].at[k]],
                                      rowb[b].at[pl.ds(k * CH, CH)],
                                      semg[b]).wait()

        def scatter(b):
            for k in range(SUB):
                pltpu.sync_copy(rowb[b].at[pl.ds(k * CH, CH)],
                                acc.at[dstb[b].at[k]], add=True)

        issue_idx(0, 0)
        wait_idx(0, 0)
        _localize_dst(dstb[0], base, half)
        issue_gathers(0)
        issue_idx(1, 1)
        wait_gathers(0)
        scatter(0)

        @pl.loop(0, (NCHB - 1) // 3)
        def _(t):
            for k in range(3):
                cc = 3 * t + 1 + k
                b = (1 + k) % 3
                bn = (2 + k) % 3
                wait_idx(cc, b)
                issue_idx(cc + 1, bn)
                _localize_dst(dstb[b], base, half)
                issue_gathers(b)
                wait_gathers(b)
                scatter(b)

        wait_idx(NCHB, NCHB % 3)

        plsc.subcore_barrier()
        _writeback(acc, out_hbm, c, s)

    return agg(y, src2d, dst2d)


def _sc_degree(dst, ones_arr):
    @functools.partial(
        pl.kernel, mesh=_sc_mesh(),
        out_type=jax.ShapeDtypeStruct((N, DEGW), jnp.float32),
        compiler_params=pltpu.CompilerParams(use_tc_tiling_on_sc=False),
        scratch_types=[
            pltpu.VMEM((SUB, CH), jnp.int32),
            pltpu.VMEM((SUB, CH), jnp.int32),
            pltpu.VMEM((CH, DEGW), jnp.float32),
            pltpu.VMEM((CH, DEGW), jnp.float32),
            pltpu.VMEM_SHARED((TRASH + 8, DEGW), jnp.float32),
            pltpu.SemaphoreType.DMA,
            pltpu.SemaphoreType.DMA,
        ])
    def deg(dst_hbm, ones_hbm, out_hbm, d0, d1, ones_v, zeros_v, acc,
            si0, si1):
        c = lax.axis_index("c")
        s = lax.axis_index("s")
        base = c * HALF0
        half = jnp.where(c == 0, HALF0, HALF1)
        dstb, semi = (d0, d1), (si0, si1)
        zero16 = jnp.zeros((16,), jnp.float32)

        pltpu.sync_copy(ones_hbm, ones_v)

        @pl.loop(0, CH)
        def _(r):
            zeros_v[r, pl.ds(0, 16)] = zero16

        @pl.loop(0, ZSTRIPE // CH)
        def _(k):
            pltpu.sync_copy(zeros_v, acc.at[pl.ds(s * ZSTRIPE + k * CH, CH)])

        _zrem = ZSTRIPE - (ZSTRIPE // CH) * CH
        if _zrem:
            pltpu.sync_copy(zeros_v.at[pl.ds(0, _zrem)],
                            acc.at[pl.ds(s * ZSTRIPE + ZSTRIPE - _zrem, _zrem)])

        @pl.when(s == 0)
        def _():
            pltpu.sync_copy(zeros_v.at[pl.ds(0, 8)], acc.at[pl.ds(ACC_ROWS, 8)])

        plsc.subcore_barrier()

        def row0(cc):
            return (s * NCHB + cc) * SUB

        def issue_idx(cc, b):
            pltpu.async_copy(dst_hbm.at[pl.ds(row0(cc), SUB)], dstb[b], semi[b])

        def wait_idx(cc, b):
            pltpu.make_async_copy(dst_hbm.at[pl.ds(row0(cc), SUB)], dstb[b],
                                  semi[b]).wait()

        def step(cc, b):
            wait_idx(cc, b)
            issue_idx(cc + 1, 1 - b)
            _localize_dst(dstb[b], base, half)
            for k in range(SUB):
                pltpu.sync_copy(ones_v, acc.at[dstb[b].at[k]], add=True)

        issue_idx(0, 0)

        @pl.loop(0, (NCHB - 1) // 2)
        def _(t):
            for b in range(2):
                step(2 * t + b, b)

        step(NCHB - 1, (NCHB - 1) % 2)
        wait_idx(NCHB, NCHB % 2)

        plsc.subcore_barrier()
        _writeback(acc, out_hbm, c, s)

    return deg(dst, ones_arr)


R = 400
NBLK = N // R

_F32 = jnp.float32


def _dot(a, b):
    return jnp.dot(a, b, preferred_element_type=_F32)


def _full(shape):
    return pl.BlockSpec(shape, lambda i: (0,) * len(shape))


def _rows(width):
    return pl.BlockSpec((R, width), lambda i: (i, 0))


def _tc_prologue(feat128, cfg, aux, W_feat, W_cfg, shape_emb, W_set,
                 op_emb, W_op, bias):
    def body(f_ref, c_ref, a_ref, wf_ref, wc_ref, se_ref, ws_ref, oe_ref,
             wo_ref, b_ref, o_ref):
        x = _dot(f_ref[...], wf_ref[...])
        x += _dot(c_ref[...], wc_ref[...])
        Se = _dot(se_ref[...], ws_ref[...])
        Oe = _dot(oe_ref[...], wo_ref[...])
        aux_v = a_ref[...]
        sidx = aux_v[:, 0:1].astype(jnp.int32)
        oidx = aux_v[:, 1:2].astype(jnp.int32)
        oh_s = (lax.broadcasted_iota(jnp.int32, (R, 8), 1) == sidx).astype(_F32)
        x += _dot(oh_s, Se)
        oh_o = (lax.broadcasted_iota(jnp.int32, (R, N_OPS), 1) == oidx).astype(_F32)
        x += _dot(oh_o, Oe)
        o_ref[...] = jnp.maximum(x + b_ref[...], 0.0)

    return pl.pallas_call(
        body,
        grid=(NBLK,),
        in_specs=[_rows(128), _rows(18), _rows(8), _full((128, 128)),
                  _full((18, 128)), _full((8, 4)), _full((4, 128)),
                  _full((N_OPS, 32)), _full((32, 128)), _full((1, 128))],
        out_specs=_rows(128),
        out_shape=jax.ShapeDtypeStruct((N, 128), _F32),
    )(feat128, cfg, aux, W_feat, W_cfg, shape_emb, W_set, op_emb, W_op, bias)


def _tc_dense1(x, Wp, bp, Wl, ind):
    def body(x_ref, wp_ref, bp_ref, wl_ref, o_ref):
        xp = jnp.maximum(_dot(x_ref[...], wp_ref[...]) + bp_ref[...], 0.0)
        o_ref[...] = _dot(xp, wl_ref[...])

    return pl.pallas_call(
        body,
        grid=(NBLK,),
        in_specs=[_rows(ind), _full((ind, ind)), _full((1, ind)),
                  _full((ind, H))],
        out_specs=_rows(H),
        out_shape=jax.ShapeDtypeStruct((N, H), _F32),
    )(x, Wp, bp, Wl)


def _tc_dense2(s, degs, x, Wr, bl, ind):
    def body(s_ref, d_ref, x_ref, wr_ref, bl_ref, o_ref):
        cnt = jnp.maximum(d_ref[...][:, 0:1], 1.0)
        o = s_ref[...] / cnt + bl_ref[...] + _dot(x_ref[...], wr_ref[...])
        nrm = jnp.sqrt(jnp.sum(o * o, axis=1, keepdims=True))
        o_ref[...] = o / jnp.maximum(nrm, 1e-12)

    return pl.pallas_call(
        body,
        grid=(NBLK,),
        in_specs=[_rows(H), _rows(DEGW), _rows(ind), _full((ind, H)),
                  _full((1, H))],
        out_specs=_rows(H),
        out_shape=jax.ShapeDtypeStruct((N, H), _F32),
    )(s, degs, x, Wr, bl)


def _tc_pool(x, aux, pp):
    NEG = -3.4e38

    def body(x_ref, a_ref, pp_ref, o_ref, mx_ref, sm_ref, ct_ref):
        i = pl.program_id(0)

        @pl.when(i == 0)
        def _():
            mx_ref[...] = jnp.full((NG, H), NEG, _F32)
            sm_ref[...] = jnp.zeros((NG, H), _F32)
            ct_ref[...] = jnp.zeros((NG, 128), _F32)

        xv = x_ref[...]
        bidx = a_ref[...][:, 2:3].astype(jnp.int32)
        oh = (lax.broadcasted_iota(jnp.int32, (R, NG), 1) == bidx).astype(_F32)
        sm_ref[...] += lax.dot_general(oh, xv, (((0,), (0,)), ((), ())),
                                       preferred_element_type=_F32)
        ct_ref[...] += lax.dot_general(oh, jnp.ones((R, 128), _F32),
                                       (((0,), (0,)), ((), ())),
                                       preferred_element_type=_F32)
        for g in range(NG):
            m = bidx == g
            xm = jnp.where(m, xv, NEG)
            gm = jnp.max(xm, axis=0, keepdims=True)
            mx_ref[g:g + 1, :] = jnp.maximum(mx_ref[g:g + 1, :], gm)

        @pl.when(i == NBLK - 1)
        def _():
            cnt = jnp.maximum(ct_ref[...][:, 0:1], 1.0)
            xg = mx_ref[...] + sm_ref[...] / cnt
            nrm = jnp.sqrt(jnp.sum(xg * xg, axis=1, keepdims=True))
            xg = xg / nrm
            wt = pp_ref[...][0:1, 0:H]
            pb = pp_ref[...][1:2, 0:1]
            res = jnp.sum(xg * wt, axis=1, keepdims=True) + pb
            o_ref[...] = jnp.broadcast_to(res, (NG, 128))

    return pl.pallas_call(
        body,
        grid=(NBLK,),
        in_specs=[_rows(H), _rows(8), _full((8, 128))],
        out_specs=pl.BlockSpec((NG, 128), lambda i: (0, 0)),
        out_shape=jax.ShapeDtypeStruct((NG, 128), _F32),
        scratch_shapes=[pltpu.VMEM((NG, H), _F32), pltpu.VMEM((NG, H), _F32),
                        pltpu.VMEM((NG, 128), _F32)],
    )(x, aux, pp)


def kernel(node_feat, node_config_feat, node_opcode, edge_index, batch,
           op_emb, shape_emb, lin_W, lin_b,
           l0_Wp, l0_bp, l0_Wl, l0_bl, l0_Wr,
           l1_Wp, l1_bp, l1_Wl, l1_bl, l1_Wr,
           l2_Wp, l2_bp, l2_Wl, l2_bl, l2_Wr,
           post_W, post_b):
    npad = EP + EPC - E
    src = jnp.concatenate([edge_index[0], jnp.zeros((npad,), jnp.int32)])
    src = src.reshape(EROWS, CH)
    dst = jnp.concatenate([edge_index[1], jnp.full((npad,), -1, jnp.int32)])
    dst = dst.reshape(EROWS, CH)
    feat128 = node_feat[:, :128]
    aux = jnp.concatenate([
        node_feat[:, 128:129],
        node_opcode.astype(_F32)[:, None],
        batch.astype(_F32)[:, None],
        jnp.zeros((N, 5), _F32),
    ], axis=1)
    W_feat = lin_W[0:128]
    W_set = lin_W[128:132]
    W_op = lin_W[132:164]
    W_cfg = lin_W[164:182]
    bias = lin_b[None, :]
    pp = jnp.zeros((8, 128), _F32)
    pp = pp.at[0, 0:H].set(post_W[:, 0])
    pp = pp.at[1, 0].set(post_b[0])

    ones_arr = jnp.zeros((CH, DEGW), _F32).at[:, 0].set(1.0)
    degs = _sc_degree(dst, ones_arr)
    x = _tc_prologue(feat128, node_config_feat, aux, W_feat, W_cfg,
                     shape_emb, W_set, op_emb, W_op, bias)

    for Wp, bp, Wl, bl, Wr, ind in (
            (l0_Wp, l0_bp, l0_Wl, l0_bl, l0_Wr, 128),
            (l1_Wp, l1_bp, l1_Wl, l1_bl, l1_Wr, 64),
            (l2_Wp, l2_bp, l2_Wl, l2_bl, l2_Wr, 64)):
        y = _tc_dense1(x, Wp, bp[None, :], Wl, ind)
        s = _sc_aggregate(y, src, dst)
        x = _tc_dense2(s, degs, x, Wr, bl[None, :], ind)

    pooled = _tc_pool(x, aux, pp)
    return pooled[:, :1]


# speedup vs baseline: 4.0455x; 1.0942x over previous
import functools

import jax
import jax.numpy as jnp
from jax import lax
from jax.experimental import pallas as pl
from jax.experimental.pallas import tpu as pltpu
from jax.experimental.pallas import tpu_sc as plsc

N = 50000
E = 800000
H = 64
NG = 16
N_OPS = 120

NSUB = 16
CH = 128
EPC = 128
SUB = EPC // CH
NCHB = 391
EP = NSUB * NCHB * EPC
EROWS = (EP + EPC) // CH
ACC_ROWS = 16 * 1600
ZSTRIPE = ACC_ROWS // NSUB
HALF0 = 25088
HALF1 = N - HALF0
TRASH = ACC_ROWS
DEGW = 16

@functools.lru_cache(maxsize=None)
def _sc_mesh():
    return plsc.VectorSubcoreMesh(core_axis_name="c", subcore_axis_name="s",
                                  num_cores=2, num_subcores=NSUB)


def _localize_dst(dst_v, base, half):
    for k in range(SUB):
        for j in range(CH // 16):
            d = dst_v[k, pl.ds(j * 16, 16)]
            loc = d - base
            ok = (loc >= 0) & (loc < half)
            dst_v[k, pl.ds(j * 16, 16)] = jnp.where(ok, loc, TRASH)


def _writeback(acc, out_hbm, c, s):
    @pl.when(c == 0)
    def _():
        pltpu.sync_copy(acc.at[pl.ds(s * 1568, 1568)],
                        out_hbm.at[pl.ds(s * 1568, 1568)])

    @pl.when((c == 1) & (s < 15))
    def _():
        pltpu.sync_copy(acc.at[pl.ds(s * 1560, 1560)],
                        out_hbm.at[pl.ds(HALF0 + s * 1560, 1560)])

    @pl.when((c == 1) & (s == 15))
    def _():
        pltpu.sync_copy(acc.at[pl.ds(15 * 1560, 1512)],
                        out_hbm.at[pl.ds(HALF0 + 15 * 1560, 1512)])


def _sc_aggregate(y, src2d, dst2d):
    @functools.partial(
        pl.kernel, mesh=_sc_mesh(),
        out_type=jax.ShapeDtypeStruct((N, H), jnp.float32),
        compiler_params=pltpu.CompilerParams(use_tc_tiling_on_sc=False),
        scratch_types=[
            pltpu.VMEM((SUB, CH), jnp.int32),
            pltpu.VMEM((SUB, CH), jnp.int32),
            pltpu.VMEM((SUB, CH), jnp.int32),
            pltpu.VMEM((SUB, CH), jnp.int32),
            pltpu.VMEM((SUB, CH), jnp.int32),
            pltpu.VMEM((SUB, CH), jnp.int32),
            pltpu.VMEM((EPC, H), jnp.float32),
            pltpu.VMEM((EPC, H), jnp.float32),
            pltpu.VMEM((EPC, H), jnp.float32),
            pltpu.VMEM_SHARED((TRASH + 8, H), jnp.float32),
            pltpu.SemaphoreType.DMA,
            pltpu.SemaphoreType.DMA,
            pltpu.SemaphoreType.DMA,
            pltpu.SemaphoreType.DMA,
            pltpu.SemaphoreType.DMA,
            pltpu.SemaphoreType.DMA,
        ])
    def agg(y_hbm, src_hbm, dst_hbm, out_hbm,
            s0, s1, s2, d0, d1, d2, r0, r1, r2, acc,
            si0, si1, si2, sg0, sg1, sg2):
        c = lax.axis_index("c")
        s = lax.axis_index("s")
        base = c * HALF0
        half = jnp.where(c == 0, HALF0, HALF1)
        srcb, dstb, rowb = (s0, s1, s2), (d0, d1, d2), (r0, r1, r2)
        semi, semg = (si0, si1, si2), (sg0, sg1, sg2)
        zero16 = jnp.zeros((16,), jnp.float32)

        @pl.loop(0, EPC)
        def _(r):
            for j in range(H // 16):
                r0[r, pl.ds(j * 16, 16)] = zero16

        for kk in range(ZSTRIPE // EPC):
            pltpu.sync_copy(r0, acc.at[pl.ds(s * ZSTRIPE + kk * EPC, EPC)])
        _zrem = ZSTRIPE - (ZSTRIPE // EPC) * EPC
        if _zrem:
            pltpu.sync_copy(r0.at[pl.ds(0, _zrem)],
                            acc.at[pl.ds(s * ZSTRIPE + ZSTRIPE - _zrem, _zrem)])

        @pl.when(s == 0)
        def _():
            pltpu.sync_copy(r0.at[pl.ds(0, 8)], acc.at[pl.ds(ACC_ROWS, 8)])

        plsc.subcore_barrier()

        def row0(cc):
            return (s * NCHB + cc) * SUB

        def issue_idx(cc, b):
            pltpu.async_copy(src_hbm.at[pl.ds(row0(cc), SUB)], srcb[b], semi[b])
            pltpu.async_copy(dst_hbm.at[pl.ds(row0(cc), SUB)], dstb[b], semi[b])

        def wait_idx(cc, b):
            pltpu.make_async_copy(src_hbm.at[pl.ds(row0(cc), SUB)], srcb[b],
                                  semi[b]).wait()
            pltpu.make_async_copy(dst_hbm.at[pl.ds(row0(cc), SUB)], dstb[b],
                                  semi[b]).wait()

        def issue_gathers(b):
            for k in range(SUB):
                pltpu.async_copy(y_hbm.at[srcb[b].at[k]],
                                 rowb[b].at[pl.ds(k * CH, CH)], semg[b])

        def wait_gathers(b):
            for k in range(SUB):
                pltpu.make_async_copy(y_hbm.at[srcb[b].at[k]],
                                      rowb[b].at[pl.ds(k * CH, CH)],
                                      semg[b]).wait()

        def scatter(b):
            for k in range(SUB):
                pltpu.sync_copy(rowb[b].at[pl.ds(k * CH, CH)],
                                acc.at[dstb[b].at[k]], add=True)

        issue_idx(0, 0)
        wait_idx(0, 0)
        _localize_dst(dstb[0], base, half)
        issue_gathers(0)
        issue_idx(1, 1)
        wait_gathers(0)
        scatter(0)

        @pl.loop(0, (NCHB - 1) // 3)
        def _(t):
            for k in range(3):
                cc = 3 * t + 1 + k
                b = (1 + k) % 3
                bn = (2 + k) % 3
                wait_idx(cc, b)
                issue_idx(cc + 1, bn)
                _localize_dst(dstb[b], base, half)
                issue_gathers(b)
                wait_gathers(b)
                scatter(b)

        wait_idx(NCHB, NCHB % 3)

        plsc.subcore_barrier()
        _writeback(acc, out_hbm, c, s)

    return agg(y, src2d, dst2d)


def _sc_degree(dst, ones_arr):
    @functools.partial(
        pl.kernel, mesh=_sc_mesh(),
        out_type=jax.ShapeDtypeStruct((N, DEGW), jnp.float32),
        compiler_params=pltpu.CompilerParams(use_tc_tiling_on_sc=False),
        scratch_types=[
            pltpu.VMEM((SUB, CH), jnp.int32),
            pltpu.VMEM((SUB, CH), jnp.int32),
            pltpu.VMEM((CH, DEGW), jnp.float32),
            pltpu.VMEM((CH, DEGW), jnp.float32),
            pltpu.VMEM_SHARED((TRASH + 8, DEGW), jnp.float32),
            pltpu.SemaphoreType.DMA,
            pltpu.SemaphoreType.DMA,
        ])
    def deg(dst_hbm, ones_hbm, out_hbm, d0, d1, ones_v, zeros_v, acc,
            si0, si1):
        c = lax.axis_index("c")
        s = lax.axis_index("s")
        base = c * HALF0
        half = jnp.where(c == 0, HALF0, HALF1)
        dstb, semi = (d0, d1), (si0, si1)
        zero16 = jnp.zeros((16,), jnp.float32)

        pltpu.sync_copy(ones_hbm, ones_v)

        @pl.loop(0, CH)
        def _(r):
            zeros_v[r, pl.ds(0, 16)] = zero16

        @pl.loop(0, ZSTRIPE // CH)
        def _(k):
            pltpu.sync_copy(zeros_v, acc.at[pl.ds(s * ZSTRIPE + k * CH, CH)])

        _zrem = ZSTRIPE - (ZSTRIPE // CH) * CH
        if _zrem:
            pltpu.sync_copy(zeros_v.at[pl.ds(0, _zrem)],
                            acc.at[pl.ds(s * ZSTRIPE + ZSTRIPE - _zrem, _zrem)])

        @pl.when(s == 0)
        def _():
            pltpu.sync_copy(zeros_v.at[pl.ds(0, 8)], acc.at[pl.ds(ACC_ROWS, 8)])

        plsc.subcore_barrier()

        def row0(cc):
            return (s * NCHB + cc) * SUB

        def issue_idx(cc, b):
            pltpu.async_copy(dst_hbm.at[pl.ds(row0(cc), SUB)], dstb[b], semi[b])

        def wait_idx(cc, b):
            pltpu.make_async_copy(dst_hbm.at[pl.ds(row0(cc), SUB)], dstb[b],
                                  semi[b]).wait()

        def step(cc, b):
            wait_idx(cc, b)
            issue_idx(cc + 1, 1 - b)
            _localize_dst(dstb[b], base, half)
            for k in range(SUB):
                pltpu.sync_copy(ones_v, acc.at[dstb[b].at[k]], add=True)

        issue_idx(0, 0)

        @pl.loop(0, (NCHB - 1) // 2)
        def _(t):
            for b in range(2):
                step(2 * t + b, b)

        step(NCHB - 1, (NCHB - 1) % 2)
        wait_idx(NCHB, NCHB % 2)

        plsc.subcore_barrier()
        _writeback(acc, out_hbm, c, s)

    return deg(dst, ones_arr)


R = 400
NBLK = N // R

_F32 = jnp.float32


def _dot(a, b):
    return jnp.dot(a, b, preferred_element_type=_F32)


def _full(shape):
    return pl.BlockSpec(shape, lambda i: (0,) * len(shape))


def _rows(width):
    return pl.BlockSpec((R, width), lambda i: (i, 0))


def _d1(x, wp_ref, bp_ref, wl_ref):
    xp = jnp.maximum(_dot(x, wp_ref[...]) + bp_ref[...], 0.0)
    return _dot(xp, wl_ref[...])


def _d2(s_ref, d_ref, x, wr_ref, bl_ref):
    cnt = jnp.maximum(d_ref[...][:, 0:1], 1.0)
    o = s_ref[...] / cnt + bl_ref[...] + _dot(x, wr_ref[...])
    nrm = jnp.sqrt(jnp.sum(o * o, axis=1, keepdims=True))
    return o / jnp.maximum(nrm, 1e-12)


def _tc_k0(feat128, cfg, aux, W_feat, W_cfg, shape_emb, W_set,
           op_emb, W_op, bias, Wp, bp, Wl):
    def body(f_ref, c_ref, a_ref, wf_ref, wc_ref, se_ref, ws_ref, oe_ref,
             wo_ref, b_ref, wp_ref, bp_ref, wl_ref, x_ref, y_ref):
        x = _dot(f_ref[...], wf_ref[...])
        x += _dot(c_ref[...], wc_ref[...])
        Se = _dot(se_ref[...], ws_ref[...])
        Oe = _dot(oe_ref[...], wo_ref[...])
        aux_v = a_ref[...]
        sidx = aux_v[:, 0:1].astype(jnp.int32)
        oidx = aux_v[:, 1:2].astype(jnp.int32)
        oh_s = (lax.broadcasted_iota(jnp.int32, (R, 8), 1) == sidx).astype(_F32)
        x += _dot(oh_s, Se)
        oh_o = (lax.broadcasted_iota(jnp.int32, (R, N_OPS), 1) == oidx).astype(_F32)
        x += _dot(oh_o, Oe)
        x = jnp.maximum(x + b_ref[...], 0.0)
        x_ref[...] = x
        y_ref[...] = _d1(x, wp_ref, bp_ref, wl_ref)

    return pl.pallas_call(
        body,
        grid=(NBLK,),
        in_specs=[_rows(128), _rows(18), _rows(8), _full((128, 128)),
                  _full((18, 128)), _full((8, 4)), _full((4, 128)),
                  _full((N_OPS, 32)), _full((32, 128)), _full((1, 128)),
                  _full((128, 128)), _full((1, 128)), _full((128, H))],
        out_specs=(_rows(128), _rows(H)),
        out_shape=(jax.ShapeDtypeStruct((N, 128), _F32),
                   jax.ShapeDtypeStruct((N, H), _F32)),
    )(feat128, cfg, aux, W_feat, W_cfg, shape_emb, W_set, op_emb, W_op,
      bias, Wp, bp, Wl)


def _tc_fused(s, degs, x, Wr, bl, Wp, bp, Wl, ind):
    def body(s_ref, d_ref, x_ref, wr_ref, bl_ref, wp_ref, bp_ref, wl_ref,
             xn_ref, y_ref):
        xn = _d2(s_ref, d_ref, x_ref[...], wr_ref, bl_ref)
        xn_ref[...] = xn
        y_ref[...] = _d1(xn, wp_ref, bp_ref, wl_ref)

    return pl.pallas_call(
        body,
        grid=(NBLK,),
        in_specs=[_rows(H), _rows(DEGW), _rows(ind), _full((ind, H)),
                  _full((1, H)), _full((H, H)), _full((1, H)),
                  _full((H, H))],
        out_specs=(_rows(H), _rows(H)),
        out_shape=(jax.ShapeDtypeStruct((N, H), _F32),
                   jax.ShapeDtypeStruct((N, H), _F32)),
    )(s, degs, x, Wr, bl, Wp, bp, Wl)


def _tc_k3(s, degs, x, Wr, bl, aux, pp, ind):
    NEG = -3.4e38

    def body(s_ref, d_ref, x_ref, wr_ref, bl_ref, a_ref, pp_ref, o_ref,
             mx_ref, sm_ref, ct_ref):
        i = pl.program_id(0)

        @pl.when(i == 0)
        def _():
            mx_ref[...] = jnp.full((NG, H), NEG, _F32)
            sm_ref[...] = jnp.zeros((NG, H), _F32)
            ct_ref[...] = jnp.zeros((NG, 128), _F32)

        xv = _d2(s_ref, d_ref, x_ref[...], wr_ref, bl_ref)
        bidx = a_ref[...][:, 2:3].astype(jnp.int32)
        oh = (lax.broadcasted_iota(jnp.int32, (R, NG), 1) == bidx).astype(_F32)
        sm_ref[...] += lax.dot_general(oh, xv, (((0,), (0,)), ((), ())),
                                       preferred_element_type=_F32)
        ct_ref[...] += lax.dot_general(oh, jnp.ones((R, 128), _F32),
                                       (((0,), (0,)), ((), ())),
                                       preferred_element_type=_F32)
        for g in range(NG):
            m = bidx == g
            xm = jnp.where(m, xv, NEG)
            gm = jnp.max(xm, axis=0, keepdims=True)
            mx_ref[g:g + 1, :] = jnp.maximum(mx_ref[g:g + 1, :], gm)

        @pl.when(i == NBLK - 1)
        def _():
            cnt = jnp.maximum(ct_ref[...][:, 0:1], 1.0)
            xg = mx_ref[...] + sm_ref[...] / cnt
            nrm = jnp.sqrt(jnp.sum(xg * xg, axis=1, keepdims=True))
            xg = xg / nrm
            wt = pp_ref[...][0:1, 0:H]
            pb = pp_ref[...][1:2, 0:1]
            res = jnp.sum(xg * wt, axis=1, keepdims=True) + pb
            o_ref[...] = jnp.broadcast_to(res, (NG, 128))

    return pl.pallas_call(
        body,
        grid=(NBLK,),
        in_specs=[_rows(H), _rows(DEGW), _rows(ind), _full((ind, H)),
                  _full((1, H)), _rows(8), _full((8, 128))],
        out_specs=pl.BlockSpec((NG, 128), lambda i: (0, 0)),
        out_shape=jax.ShapeDtypeStruct((NG, 128), _F32),
        scratch_shapes=[pltpu.VMEM((NG, H), _F32), pltpu.VMEM((NG, H), _F32),
                        pltpu.VMEM((NG, 128), _F32)],
    )(s, degs, x, Wr, bl, aux, pp)


def kernel(node_feat, node_config_feat, node_opcode, edge_index, batch,
           op_emb, shape_emb, lin_W, lin_b,
           l0_Wp, l0_bp, l0_Wl, l0_bl, l0_Wr,
           l1_Wp, l1_bp, l1_Wl, l1_bl, l1_Wr,
           l2_Wp, l2_bp, l2_Wl, l2_bl, l2_Wr,
           post_W, post_b):
    npad = EP + EPC - E
    src = jnp.concatenate([edge_index[0], jnp.zeros((npad,), jnp.int32)])
    src = src.reshape(EROWS, CH)
    dst = jnp.concatenate([edge_index[1], jnp.full((npad,), -1, jnp.int32)])
    dst = dst.reshape(EROWS, CH)
    feat128 = node_feat[:, :128]
    aux = jnp.concatenate([
        node_feat[:, 128:129],
        node_opcode.astype(_F32)[:, None],
        batch.astype(_F32)[:, None],
        jnp.zeros((N, 5), _F32),
    ], axis=1)
    W_feat = lin_W[0:128]
    W_set = lin_W[128:132]
    W_op = lin_W[132:164]
    W_cfg = lin_W[164:182]
    bias = lin_b[None, :]
    pp = jnp.zeros((8, 128), _F32)
    pp = pp.at[0, 0:H].set(post_W[:, 0])
    pp = pp.at[1, 0].set(post_b[0])

    ones_arr = jnp.zeros((CH, DEGW), _F32).at[:, 0].set(1.0)
    degs = _sc_degree(dst, ones_arr)

    x0, y0 = _tc_k0(feat128, node_config_feat, aux, W_feat, W_cfg,
                    shape_emb, W_set, op_emb, W_op, bias,
                    l0_Wp, l0_bp[None, :], l0_Wl)
    s0 = _sc_aggregate(y0, src, dst)
    x1, y1 = _tc_fused(s0, degs, x0, l0_Wr, l0_bl[None, :],
                       l1_Wp, l1_bp[None, :], l1_Wl, 128)
    s1 = _sc_aggregate(y1, src, dst)
    x2, y2 = _tc_fused(s1, degs, x1, l1_Wr, l1_bl[None, :],
                       l2_Wp, l2_bp[None, :], l2_Wl, 64)
    s2 = _sc_aggregate(y2, src, dst)
    pooled = _tc_k3(s2, degs, x2, l2_Wr, l2_bl[None, :], aux, pp, 64)
    return pooled[:, :1]
